# baseline (speedup 1.0000x reference)
import functools
import math

import jax
import jax.numpy as jnp
from jax import lax
from jax.experimental import pallas as pl
from jax.experimental.pallas import tpu as pltpu
from jax.experimental.pallas import tpu_sc as plsc

_NUM_BUCKETS = 32
_MAX_DISTANCE = 512
_SEQ = 2048
_H = 12
_NSHIFT = 16
_DW = 4352
_DP = 4480
_NC = 2
_NS = 16
_NW = _NC * _NS
_ROWS_PER_W = _SEQ // _NW


def _bucket_of(rel):
    nb2 = _NUM_BUCKETS // 2
    me = _NUM_BUCKETS // 4
    rb = (rel > 0).astype(jnp.int32) * nb2
    rb = rb + (rel < 0).astype(jnp.int32) * nb2
    rp = jnp.abs(rel)
    is_small = rp < me
    rp_safe = jnp.maximum(rp, 1).astype(jnp.float32)
    rp_if_large = me + (
        jnp.log(rp_safe / me) / math.log(_MAX_DISTANCE / me) * (nb2 - me)
    ).astype(jnp.int32)
    rp_if_large = jnp.minimum(rp_if_large, nb2 - 1)
    return rb + jnp.where(is_small, rp, rp_if_large)


def _build_body(table_t_ref, d16_ref):
    u = jax.lax.broadcasted_iota(jnp.int32, (1, _DP), 1)
    bucket = _bucket_of(u - (_SEQ - 1))
    b_iota = jax.lax.broadcasted_iota(jnp.int32, (_NUM_BUCKETS, _DP), 0)
    onehot = (b_iota == bucket).astype(jnp.float32)
    dvpad = jax.lax.dot_general(
        table_t_ref[...], onehot, (((1,), (0,)), ((), ())),
        preferred_element_type=jnp.float32)
    for s in range(_NSHIFT):
        d16_ref[s] = pltpu.roll(dvpad, (-s) % _DP, axis=1)[:, :_DW]


def _build_d16(table_t):
    return pl.pallas_call(
        _build_body,
        in_specs=[pl.BlockSpec((_H, _NUM_BUCKETS), lambda: (0, 0))],
        out_specs=pl.BlockSpec((_NSHIFT, _H, _DW), lambda: (0, 0, 0)),
        out_shape=jax.ShapeDtypeStruct((_NSHIFT, _H, _DW), jnp.float32),
    )(table_t)


def _expand_body(d16_hbm, out_hbm):
    wid = lax.axis_index("s") * _NC + lax.axis_index("c")
    base_i = wid * _ROWS_PER_W

    def body(t, carry):
        r = t // _H
        h = t - r * _H
        i = base_i + r
        off = (_SEQ - 1) - i
        s = lax.rem(off, _NSHIFT)
        q = lax.div(off, _NSHIFT)
        pltpu.sync_copy(d16_hbm.at[s, h, pl.ds(_NSHIFT * q, _SEQ)],
                        out_hbm.at[h, i])
        return carry

    lax.fori_loop(0, _ROWS_PER_W * _H, body, 0)


def kernel(relative_attention_bias, seq_length):
    del seq_length
    table_t = relative_attention_bias.T
    d16 = _build_d16(table_t)
    mesh = plsc.VectorSubcoreMesh(
        core_axis_name="c", subcore_axis_name="s",
        num_cores=_NC, num_subcores=_NS)
    expand = functools.partial(
        pl.kernel,
        out_type=jax.ShapeDtypeStruct((_H, _SEQ, _SEQ), jnp.float32),
        mesh=mesh,
        compiler_params=pltpu.CompilerParams(use_tc_tiling_on_sc=False),
    )(_expand_body)
    return expand(d16)


# speedup vs baseline: 1.0024x; 1.0024x over previous
import functools
import math

import jax
import jax.numpy as jnp
from jax import lax
from jax.experimental import pallas as pl
from jax.experimental.pallas import tpu as pltpu
from jax.experimental.pallas import tpu_sc as plsc

_NUM_BUCKETS = 32
_MAX_DISTANCE = 512
_SEQ = 2048
_H = 12
_NSHIFT = 16
_DW = 4352
_DP = 4480
_NC = 2
_NS = 16
_NW = _NC * _NS
_ROWS_PER_W = _SEQ // _NW


def _bucket_of(rel):
    nb2 = _NUM_BUCKETS // 2
    me = _NUM_BUCKETS // 4
    rb = (rel > 0).astype(jnp.int32) * nb2
    rb = rb + (rel < 0).astype(jnp.int32) * nb2
    rp = jnp.abs(rel)
    is_small = rp < me
    rp_safe = jnp.maximum(rp, 1).astype(jnp.float32)
    rp_if_large = me + (
        jnp.log(rp_safe / me) / math.log(_MAX_DISTANCE / me) * (nb2 - me)
    ).astype(jnp.int32)
    rp_if_large = jnp.minimum(rp_if_large, nb2 - 1)
    return rb + jnp.where(is_small, rp, rp_if_large)


def _build_body(table_t_ref, d16_ref):
    u = jax.lax.broadcasted_iota(jnp.int32, (1, _DP), 1)
    bucket = _bucket_of(u - (_SEQ - 1))
    b_iota = jax.lax.broadcasted_iota(jnp.int32, (_NUM_BUCKETS, _DP), 0)
    onehot = (b_iota == bucket).astype(jnp.float32)
    dvpad = jax.lax.dot_general(
        table_t_ref[...], onehot, (((1,), (0,)), ((), ())),
        preferred_element_type=jnp.float32)
    for s in range(_NSHIFT):
        d16_ref[s] = pltpu.roll(dvpad, (-s) % _DP, axis=1)[:, :_DW]


def _build_d16(table_t):
    return pl.pallas_call(
        _build_body,
        in_specs=[pl.BlockSpec((_H, _NUM_BUCKETS), lambda: (0, 0))],
        out_specs=pl.BlockSpec((_NSHIFT, _H, _DW), lambda: (0, 0, 0)),
        out_shape=jax.ShapeDtypeStruct((_NSHIFT, _H, _DW), jnp.float32),
    )(table_t)


_K = 4


def _expand_body(d16_hbm, out_hbm, sem):
    wid = lax.axis_index("s") * _NC + lax.axis_index("c")
    base_i = wid * _ROWS_PER_W

    def body(r, carry):
        i = base_i + r
        off = (_SEQ - 1) - i
        s = lax.rem(off, _NSHIFT)
        q = lax.div(off, _NSHIFT)
        pltpu.async_copy(d16_hbm.at[s, :, pl.ds(_NSHIFT * q, _SEQ)],
                         out_hbm.at[:, i, :], sem)

        @pl.when(r >= _K)
        def _drain_one():
            pltpu.make_async_copy(d16_hbm.at[0, :, pl.ds(0, _SEQ)],
                                  out_hbm.at[:, 0, :], sem).wait()

        return carry

    lax.fori_loop(0, _ROWS_PER_W, body, 0)
    for _ in range(_K):
        pltpu.make_async_copy(d16_hbm.at[0, :, pl.ds(0, _SEQ)],
                              out_hbm.at[:, 0, :], sem).wait()


def kernel(relative_attention_bias, seq_length):
    del seq_length
    table_t = relative_attention_bias.T
    d16 = _build_d16(table_t)
    mesh = plsc.VectorSubcoreMesh(
        core_axis_name="c", subcore_axis_name="s",
        num_cores=_NC, num_subcores=_NS)
    expand = functools.partial(
        pl.kernel,
        out_type=jax.ShapeDtypeStruct((_H, _SEQ, _SEQ), jnp.float32),
        mesh=mesh,
        scratch_types=[pltpu.SemaphoreType.DMA],
        compiler_params=pltpu.CompilerParams(use_tc_tiling_on_sc=False),
    )(_expand_body)
    return expand(d16)


# speedup vs baseline: 17.6100x; 17.5675x over previous
import functools
import math

import jax
import jax.numpy as jnp
from jax import lax
from jax.experimental import pallas as pl
from jax.experimental.pallas import tpu as pltpu
from jax.experimental.pallas import tpu_sc as plsc

_NUM_BUCKETS = 32
_MAX_DISTANCE = 512
_SEQ = 2048
_H = 12
_NSHIFT = 16
_DW = 4352
_DP = 4480
_NC = 2
_NS = 16
_NW = _NC * _NS
_ROWS_PER_W = _SEQ // _NW


def _bucket_of(rel):
    nb2 = _NUM_BUCKETS // 2
    me = _NUM_BUCKETS // 4
    rb = (rel > 0).astype(jnp.int32) * nb2
    rb = rb + (rel < 0).astype(jnp.int32) * nb2
    rp = jnp.abs(rel)
    is_small = rp < me
    rp_safe = jnp.maximum(rp, 1).astype(jnp.float32)
    rp_if_large = me + (
        jnp.log(rp_safe / me) / math.log(_MAX_DISTANCE / me) * (nb2 - me)
    ).astype(jnp.int32)
    rp_if_large = jnp.minimum(rp_if_large, nb2 - 1)
    return rb + jnp.where(is_small, rp, rp_if_large)


def _build_body(table_t_ref, d16_ref):
    u = jax.lax.broadcasted_iota(jnp.int32, (1, _DP), 1)
    bucket = _bucket_of(u - (_SEQ - 1))
    b_iota = jax.lax.broadcasted_iota(jnp.int32, (_NUM_BUCKETS, _DP), 0)
    onehot = (b_iota == bucket).astype(jnp.float32)
    dvpad = jax.lax.dot_general(
        table_t_ref[...], onehot, (((1,), (0,)), ((), ())),
        preferred_element_type=jnp.float32)
    for s in range(_NSHIFT):
        d16_ref[s] = pltpu.roll(dvpad, (-s) % _DP, axis=1)[:, :_DW]


def _build_d16(table_t):
    return pl.pallas_call(
        _build_body,
        in_specs=[pl.BlockSpec((_H, _NUM_BUCKETS), lambda: (0, 0))],
        out_specs=pl.BlockSpec((_NSHIFT, _H, _DW), lambda: (0, 0, 0)),
        out_shape=jax.ShapeDtypeStruct((_NSHIFT, _H, _DW), jnp.float32),
    )(table_t)


_NBUF = 5
_PF = 4


def _expand_body(d16_hbm, out_hbm, *scratch):
    bufs = scratch[:_NBUF]
    sems_in = scratch[_NBUF:2 * _NBUF]
    sems_out = scratch[2 * _NBUF:]
    wid = lax.axis_index("s") * _NC + lax.axis_index("c")
    base_i = wid * _ROWS_PER_W

    def fire_gather(r, b):
        i = base_i + r
        off = (_SEQ - 1) - i
        s = lax.rem(off, _NSHIFT)
        q = lax.div(off, _NSHIFT)
        pltpu.async_copy(d16_hbm.at[s, :, pl.ds(_NSHIFT * q, _SEQ)],
                         bufs[b], sems_in[b])

    def fire_scatter(r, b):
        i = base_i + r
        pltpu.async_copy(bufs[b], out_hbm.at[:, i, :], sems_out[b])

    def wait_in(b):
        pltpu.make_async_copy(d16_hbm.at[0, :, pl.ds(0, _SEQ)],
                              bufs[b], sems_in[b]).wait()

    def wait_out(b):
        pltpu.make_async_copy(bufs[b], out_hbm.at[:, 0, :],
                              sems_out[b]).wait()

    for b in range(_PF):
        fire_gather(b, b)

    n_groups = (_ROWS_PER_W + _NBUF - 1) // _NBUF

    def body(g, carry):
        r0 = g * _NBUF
        for b in range(_NBUF):
            r = r0 + b

            @pl.when(r < _ROWS_PER_W)
            def _scat():
                wait_in(b)
                fire_scatter(r, b)

            bn = (b + _PF) % _NBUF

            @pl.when(r + _PF < _ROWS_PER_W)
            def _prefetch():
                @pl.when(r >= 1)
                def _free():
                    wait_out(bn)

                fire_gather(r + _PF, bn)

        return carry

    lax.fori_loop(0, n_groups, body, 0)
    for b in range(_NBUF):
        wait_out(b)


def kernel(relative_attention_bias, seq_length):
    del seq_length
    table_t = relative_attention_bias.T
    d16 = _build_d16(table_t)
    mesh = plsc.VectorSubcoreMesh(
        core_axis_name="c", subcore_axis_name="s",
        num_cores=_NC, num_subcores=_NS)
    expand = functools.partial(
        pl.kernel,
        out_type=jax.ShapeDtypeStruct((_H, _SEQ, _SEQ), jnp.float32),
        mesh=mesh,
        scratch_types=(
            [pltpu.VMEM((_H, _SEQ), jnp.float32)] * _NBUF
            + [pltpu.SemaphoreType.DMA] * (2 * _NBUF)
        ),
        compiler_params=pltpu.CompilerParams(use_tc_tiling_on_sc=False),
    )(_expand_body)
    return expand(d16)


# speedup vs baseline: 36.1523x; 2.0529x over previous
import functools
import math

import jax
import jax.numpy as jnp
from jax import lax
from jax.experimental import pallas as pl
from jax.experimental.pallas import tpu as pltpu
from jax.experimental.pallas import tpu_sc as plsc

_NUM_BUCKETS = 32
_MAX_DISTANCE = 512
_SEQ = 2048
_H = 12
_NSHIFT = 128
_DW = 4352
_DP = 4480
_NC = 2
_NS = 16
_NW = _NC * _NS
_ROWS_PER_W = _SEQ // _NW
_G = 8
_W = 512
_NCHUNK = _SEQ // _W
_CHUNKS = (_ROWS_PER_W // _G) * _NCHUNK


def _bucket_of(rel):
    nb2 = _NUM_BUCKETS // 2
    me = _NUM_BUCKETS // 4
    rb = (rel > 0).astype(jnp.int32) * nb2
    rb = rb + (rel < 0).astype(jnp.int32) * nb2
    rp = jnp.abs(rel)
    is_small = rp < me
    rp_safe = jnp.maximum(rp, 1).astype(jnp.float32)
    rp_if_large = me + (
        jnp.log(rp_safe / me) / math.log(_MAX_DISTANCE / me) * (nb2 - me)
    ).astype(jnp.int32)
    rp_if_large = jnp.minimum(rp_if_large, nb2 - 1)
    return rb + jnp.where(is_small, rp, rp_if_large)


def _build_body(table_t_ref, d128_ref):
    u = jax.lax.broadcasted_iota(jnp.int32, (1, _DP), 1)
    bucket = _bucket_of(u - (_SEQ - 1))
    b_iota = jax.lax.broadcasted_iota(jnp.int32, (_NUM_BUCKETS, _DP), 0)
    onehot = (b_iota == bucket).astype(jnp.float32)
    dvpad = jax.lax.dot_general(
        table_t_ref[...], onehot, (((1,), (0,)), ((), ())),
        preferred_element_type=jnp.float32)
    for s in range(_NSHIFT):
        d128_ref[s] = pltpu.roll(dvpad, (-s) % _DP, axis=1)[:, :_DW]


def _build_d128(table_t):
    return pl.pallas_call(
        _build_body,
        in_specs=[pl.BlockSpec((_H, _NUM_BUCKETS), lambda: (0, 0))],
        out_specs=pl.BlockSpec((_NSHIFT, _H, _DW), lambda: (0, 0, 0)),
        out_shape=jax.ShapeDtypeStruct((_NSHIFT, _H, _DW), jnp.float32),
    )(table_t)


def _expand_body(d128_hbm, out_hbm, buf0, buf1, sin0, sin1, sout0, sout1):
    bufs = (buf0, buf1)
    sins = (sin0, sin1)
    souts = (sout0, sout1)
    wid = lax.axis_index("s") * _NC + lax.axis_index("c")
    base_i = wid * _ROWS_PER_W

    def fire_gathers(c, b):
        g = c // _NCHUNK
        w = c - g * _NCHUNK
        i0 = base_i + g * _G
        for k in range(_G):
            off = (_SEQ - 1) - (i0 + k)
            s = lax.rem(off, _NSHIFT)
            q = lax.div(off, _NSHIFT)
            pltpu.async_copy(
                d128_hbm.at[s, :, pl.ds(_NSHIFT * q + _W * w, _W)],
                bufs[b].at[:, k, :], sins[b])

    def wait_in(b):
        for _ in range(_G):
            pltpu.make_async_copy(d128_hbm.at[0, :, pl.ds(0, _W)],
                                  bufs[b].at[:, 0, :], sins[b]).wait()

    def fire_scatter(c, b):
        g = c // _NCHUNK
        w = c - g * _NCHUNK
        i0 = base_i + g * _G
        pltpu.async_copy(bufs[b],
                         out_hbm.at[:, pl.ds(i0, _G), pl.ds(_W * w, _W)],
                         souts[b])

    def wait_out(b):
        pltpu.make_async_copy(bufs[b],
                              out_hbm.at[:, pl.ds(0, _G), pl.ds(0, _W)],
                              souts[b]).wait()

    fire_gathers(0, 0)

    def body(c2, carry):
        for par in range(2):
            c = 2 * c2 + par
            b = par
            wait_in(b)
            fire_scatter(c, b)

            @pl.when(c + 1 < _CHUNKS)
            def _prefetch():
                @pl.when(c >= 1)
                def _free():
                    wait_out(1 - b)

                fire_gathers(c + 1, 1 - b)

        return carry

    lax.fori_loop(0, _CHUNKS // 2, body, 0)
    wait_out(0)
    wait_out(1)


def kernel(relative_attention_bias, seq_length):
    del seq_length
    table_t = relative_attention_bias.T
    d128 = _build_d128(table_t)
    mesh = plsc.VectorSubcoreMesh(
        core_axis_name="c", subcore_axis_name="s",
        num_cores=_NC, num_subcores=_NS)
    expand = functools.partial(
        pl.kernel,
        out_type=jax.ShapeDtypeStruct((_H, _SEQ, _SEQ), jnp.float32),
        mesh=mesh,
        scratch_types=(
            [pltpu.VMEM((_H, _G, _W), jnp.float32)] * 2
            + [pltpu.SemaphoreType.DMA] * 4
        ),
    )(_expand_body)
    return expand(d128)
